# baseline (speedup 1.0000x reference)
import dataclasses
import functools

import jax
import jax.numpy as jnp
from jax import lax
from jax.experimental import pallas as pl
from jax.experimental.pallas import tpu as pltpu
from jax.experimental.pallas import tpu_sc as plsc

N = 10000
E = 320000
D = 128
H = 128
OUT = 128
G = 64

NC = 2
NS = 16
NW = NC * NS
LANES = 16

C = 128
CPT = 80
EPT = CPT * C
EP = EPT * NW
NP = 10240
RPT = NP // NS
NL = 10016

BR = 1000
NBLK = N // BR

_mesh = plsc.VectorSubcoreMesh(core_axis_name="c", subcore_axis_name="s")

_sc_params = pltpu.CompilerParams()
if "needs_layout_passes" in pltpu.CompilerParams.__dataclass_fields__:
    _sc_params = dataclasses.replace(_sc_params, needs_layout_passes=False)

F32 = jnp.float32
HIGH = lax.Precision.HIGHEST


@jax.jit
def _degree_partials(dst2d):
    @functools.partial(
        pl.kernel,
        out_type=jax.ShapeDtypeStruct((NW, NL), F32),
        mesh=_mesh,
        compiler_params=_sc_params,
        scratch_types=[
            pltpu.VMEM((CPT, C), jnp.int32),
            pltpu.VMEM((NL,), F32),
        ],
    )
    def deg_kernel(dst_hbm, out_hbm, idx_v, hist_v):
        cid = lax.axis_index("c")
        sid = lax.axis_index("s")
        wid = cid * NS + sid
        zeros16 = jnp.zeros((LANES,), F32)
        ones16 = jnp.ones((LANES,), F32)

        pltpu.sync_copy(dst_hbm.at[pl.ds(wid * CPT, CPT)], idx_v)

        @pl.loop(0, NL, step=LANES)
        def _(i):
            hist_v[pl.ds(i, LANES)] = zeros16

        @pl.loop(0, CPT)
        def _(c):
            for j in range(C // LANES):
                iv = idx_v[c, pl.ds(j * LANES, LANES)]
                plsc.addupdate_scatter(hist_v, [iv], ones16)

        pltpu.sync_copy(hist_v, out_hbm.at[wid])

    return deg_kernel(dst2d)


@jax.jit
def _edge_scatter(y, src2d, dst2d):
    @functools.partial(
        pl.kernel,
        out_type=jax.ShapeDtypeStruct((NC, NP, D), F32),
        mesh=_mesh,
        compiler_params=_sc_params,
        scratch_types=[
            pltpu.VMEM((C,), jnp.int32),
            pltpu.VMEM((C,), jnp.int32),
            pltpu.VMEM((C,), jnp.int32),
            pltpu.VMEM((C,), jnp.int32),
            pltpu.VMEM((C, D), F32),
            pltpu.VMEM((C, D), F32),
            pltpu.VMEM((16, D), F32),
            pltpu.VMEM_SHARED((NP, D), F32),
            pltpu.SemaphoreType.DMA,
            pltpu.SemaphoreType.DMA,
        ],
    )
    def scat_kernel(y_hbm, src_hbm, dst_hbm, out_hbm,
                    srcv0, dstv0, srcv1, dstv1, rows0, rows1, zbuf, acc,
                    sem0, sem1):
        cid = lax.axis_index("c")
        sid = lax.axis_index("s")
        wid = cid * NS + sid
        base = wid * CPT
        zeros16 = jnp.zeros((LANES,), F32)

        pltpu.sync_copy(src_hbm.at[base], srcv0)
        pltpu.sync_copy(dst_hbm.at[base], dstv0)
        pltpu.async_copy(y_hbm.at[srcv0], rows0, sem0)

        @pl.loop(0, 16)
        def _(r):
            for j in range(D // LANES):
                zbuf[r, pl.ds(j * LANES, LANES)] = zeros16

        @pl.loop(0, RPT // 16)
        def _(k):
            pltpu.sync_copy(zbuf, acc.at[pl.ds(sid * RPT + k * 16, 16)])

        plsc.subcore_barrier()

        @pl.loop(0, CPT, step=2)
        def _(c):
            @pl.when(c + 1 < CPT)
            def _():
                pltpu.sync_copy(src_hbm.at[base + c + 1], srcv1)
                pltpu.sync_copy(dst_hbm.at[base + c + 1], dstv1)
                pltpu.async_copy(y_hbm.at[srcv1], rows1, sem1)

            pltpu.make_async_copy(y_hbm.at[srcv0], rows0, sem0).wait()
            pltpu.sync_copy(rows0, acc.at[dstv0], add=True)

            @pl.when(c + 2 < CPT)
            def _():
                pltpu.sync_copy(src_hbm.at[base + c + 2], srcv0)
                pltpu.sync_copy(dst_hbm.at[base + c + 2], dstv0)
                pltpu.async_copy(y_hbm.at[srcv0], rows0, sem0)

            @pl.when(c + 1 < CPT)
            def _():
                pltpu.make_async_copy(y_hbm.at[srcv1], rows1, sem1).wait()
                pltpu.sync_copy(rows1, acc.at[dstv1], add=True)

        plsc.subcore_barrier()
        pltpu.sync_copy(acc.at[pl.ds(sid * RPT, RPT)],
                        out_hbm.at[cid, pl.ds(sid * RPT, RPT)])

    return scat_kernel(y, src2d, dst2d)


def _dinv_of(hist_blk):
    deg = jnp.sum(hist_blk, axis=1) + 1.0
    return lax.rsqrt(deg).reshape(-1, 1)


def _prep_body(x_ref, hist_ref, w1_ref, b1_ref, w2_ref, b2_ref, gw_ref, y_ref):
    dinv = _dinv_of(hist_ref[...])
    h = jnp.maximum(
        jnp.dot(x_ref[...], w1_ref[...], preferred_element_type=F32,
                precision=HIGH) + b1_ref[...], 0.0)
    h = jnp.dot(h, w2_ref[...], preferred_element_type=F32,
                precision=HIGH) + b2_ref[...]
    xw = jnp.dot(h, gw_ref[...], preferred_element_type=F32, precision=HIGH)
    y_ref[...] = xw * dinv


@jax.jit
def _prep(x, hist, w1, b1, w2, b2, gw):
    return pl.pallas_call(
        _prep_body,
        grid=(NBLK,),
        in_specs=[
            pl.BlockSpec((BR, D), lambda i: (i, 0)),
            pl.BlockSpec((BR, NW), lambda i: (i, 0)),
            pl.BlockSpec((D, H), lambda i: (0, 0)),
            pl.BlockSpec((1, H), lambda i: (0, 0)),
            pl.BlockSpec((H, H), lambda i: (0, 0)),
            pl.BlockSpec((1, H), lambda i: (0, 0)),
            pl.BlockSpec((H, H), lambda i: (0, 0)),
        ],
        out_specs=pl.BlockSpec((BR, H), lambda i: (i, 0)),
        out_shape=jax.ShapeDtypeStruct((N, H), F32),
    )(x, hist, w1, b1, w2, b2, gw)


def _mid_body(acc_ref, y_ref, hist_ref, b_ref, gw_ref, y2_ref):
    dinv = _dinv_of(hist_ref[...])
    s = acc_ref[0] + acc_ref[1] + y_ref[...]
    h = jnp.maximum(s * dinv + b_ref[...], 0.0)
    y2_ref[...] = jnp.dot(h, gw_ref[...], preferred_element_type=F32,
                          precision=HIGH) * dinv


@jax.jit
def _mid(acc, y, hist, b, gw):
    return pl.pallas_call(
        _mid_body,
        grid=(NBLK,),
        in_specs=[
            pl.BlockSpec((NC, BR, H), lambda i: (0, i, 0)),
            pl.BlockSpec((BR, H), lambda i: (i, 0)),
            pl.BlockSpec((BR, NW), lambda i: (i, 0)),
            pl.BlockSpec((1, H), lambda i: (0, 0)),
            pl.BlockSpec((H, H), lambda i: (0, 0)),
        ],
        out_specs=pl.BlockSpec((BR, H), lambda i: (i, 0)),
        out_shape=jax.ShapeDtypeStruct((N, H), F32),
    )(acc, y, hist, b, gw)


def _final_body(acc_ref, y_ref, hist_ref, batch_ref, b_ref,
                dw1_ref, db1_ref, dw2_ref, db2_ref, out_ref, pool_scr):
    i = pl.program_id(0)
    dinv = _dinv_of(hist_ref[...])
    s = acc_ref[0] + acc_ref[1] + y_ref[...]
    h = jnp.maximum(s * dinv + b_ref[...], 0.0)
    b = batch_ref[0, 0]
    oh = (b[:, None] == lax.broadcasted_iota(jnp.int32, (BR, G), 1)).astype(F32)
    part = lax.dot_general(oh, h, (((0,), (0,)), ((), ())),
                           preferred_element_type=F32, precision=HIGH)

    @pl.when(i == 0)
    def _():
        pool_scr[...] = part

    @pl.when(i > 0)
    def _():
        pool_scr[...] += part

    @pl.when(i == NBLK - 1)
    def _():
        pooled = pool_scr[...]
        d = jnp.maximum(
            jnp.dot(pooled, dw1_ref[...], preferred_element_type=F32,
                    precision=HIGH) + db1_ref[...], 0.0)
        out_ref[...] = jnp.dot(d, dw2_ref[...], preferred_element_type=F32,
                               precision=HIGH) + db2_ref[...]


@jax.jit
def _final(acc, y, hist, batch3, b, dw1, db1, dw2, db2):
    return pl.pallas_call(
        _final_body,
        grid=(NBLK,),
        in_specs=[
            pl.BlockSpec((NC, BR, H), lambda i: (0, i, 0)),
            pl.BlockSpec((BR, H), lambda i: (i, 0)),
            pl.BlockSpec((BR, NW), lambda i: (i, 0)),
            pl.BlockSpec((1, 1, BR), lambda i: (i, 0, 0)),
            pl.BlockSpec((1, H), lambda i: (0, 0)),
            pl.BlockSpec((H, H), lambda i: (0, 0)),
            pl.BlockSpec((1, H), lambda i: (0, 0)),
            pl.BlockSpec((H, OUT), lambda i: (0, 0)),
            pl.BlockSpec((1, OUT), lambda i: (0, 0)),
        ],
        out_specs=pl.BlockSpec((G, OUT), lambda i: (0, 0)),
        out_shape=jax.ShapeDtypeStruct((G, OUT), F32),
        scratch_shapes=[pltpu.VMEM((G, H), F32)],
    )(acc, y, hist, batch3, b, dw1, db1, dw2, db2)


@jax.jit
def kernel(x, edge_index, batch, enc_W1, enc_b1, enc_W2, enc_b2,
           gW0, gb0, gW1, gb1, dec_W1, dec_b1, dec_W2, dec_b2):
    src = edge_index[0]
    dst = edge_index[1]
    pad = EP - E
    src2d = jnp.concatenate(
        [src, jnp.zeros((pad,), jnp.int32)]).reshape(NW * CPT, C)
    dst2d = jnp.concatenate(
        [dst, N + (jnp.arange(pad, dtype=jnp.int32) % C)]).reshape(NW * CPT, C)
    batch3 = batch.reshape(NBLK, 1, BR)

    hist = _degree_partials(dst2d).T

    y1 = _prep(x, hist, enc_W1, enc_b1.reshape(1, H), enc_W2,
               enc_b2.reshape(1, H), gW0)
    acc1 = _edge_scatter(y1, src2d, dst2d)
    y2 = _mid(acc1, y1, hist, gb0.reshape(1, H), gW1)
    acc2 = _edge_scatter(y2, src2d, dst2d)
    return _final(acc2, y2, hist, batch3, gb1.reshape(1, H),
                  dec_W1, dec_b1.reshape(1, H), dec_W2, dec_b2.reshape(1, OUT))


# speedup vs baseline: 2.4198x; 2.4198x over previous
import dataclasses
import functools

import jax
import jax.numpy as jnp
from jax import lax
from jax.experimental import pallas as pl
from jax.experimental.pallas import tpu as pltpu
from jax.experimental.pallas import tpu_sc as plsc

N = 10000
E = 320000
D = 128
H = 128
OUT = 128
G = 64

NC = 2
NS = 16
NW = NC * NS
LANES = 16

C = 128
CPT = 80
EPT = CPT * C
EP = EPT * NW
NP = 10240
RPT = NP // NS
NL = 10016

BR = 1000
NBLK = N // BR

_mesh = plsc.VectorSubcoreMesh(core_axis_name="c", subcore_axis_name="s")

_sc_params = pltpu.CompilerParams()
if "needs_layout_passes" in pltpu.CompilerParams.__dataclass_fields__:
    _sc_params = dataclasses.replace(_sc_params, needs_layout_passes=False)

F32 = jnp.float32
HIGH = lax.Precision.HIGHEST


@jax.jit
def _degree_partials(dst2d):
    @functools.partial(
        pl.kernel,
        out_type=jax.ShapeDtypeStruct((NW, NL), F32),
        mesh=_mesh,
        compiler_params=_sc_params,
        scratch_types=[
            pltpu.VMEM((CPT, C), jnp.int32),
            pltpu.VMEM((NL,), F32),
        ],
    )
    def deg_kernel(dst_hbm, out_hbm, idx_v, hist_v):
        cid = lax.axis_index("c")
        sid = lax.axis_index("s")
        wid = cid * NS + sid
        zeros16 = jnp.zeros((LANES,), F32)
        ones16 = jnp.ones((LANES,), F32)

        pltpu.sync_copy(dst_hbm.at[pl.ds(wid * CPT, CPT)], idx_v)

        @pl.loop(0, NL, step=LANES)
        def _(i):
            hist_v[pl.ds(i, LANES)] = zeros16

        @pl.loop(0, CPT)
        def _(c):
            for j in range(C // LANES):
                iv = idx_v[c, pl.ds(j * LANES, LANES)]
                plsc.addupdate_scatter(hist_v, [iv], ones16)

        pltpu.sync_copy(hist_v, out_hbm.at[wid])

    return deg_kernel(dst2d)


@jax.jit
def _edge_scatter(y, src2d, dst2d):
    @functools.partial(
        pl.kernel,
        out_type=jax.ShapeDtypeStruct((NC, NP, D), F32),
        mesh=_mesh,
        compiler_params=_sc_params,
        scratch_types=[
            pltpu.VMEM((C,), jnp.int32),
            pltpu.VMEM((C,), jnp.int32),
            pltpu.VMEM((C,), jnp.int32),
            pltpu.VMEM((C,), jnp.int32),
            pltpu.VMEM((C, D), F32),
            pltpu.VMEM((C, D), F32),
            pltpu.VMEM((16, D), F32),
            pltpu.VMEM_SHARED((NP, D), F32),
            pltpu.SemaphoreType.DMA,
            pltpu.SemaphoreType.DMA,
        ],
    )
    def scat_kernel(y_hbm, src_hbm, dst_hbm, out_hbm,
                    srcv0, dstv0, srcv1, dstv1, rows0, rows1, zbuf, acc,
                    sem0, sem1):
        cid = lax.axis_index("c")
        sid = lax.axis_index("s")
        wid = cid * NS + sid
        base = wid * CPT
        zeros16 = jnp.zeros((LANES,), F32)

        pltpu.sync_copy(src_hbm.at[base], srcv0)
        pltpu.sync_copy(dst_hbm.at[base], dstv0)
        pltpu.async_copy(y_hbm.at[srcv0], rows0, sem0)

        @pl.loop(0, 16)
        def _(r):
            for j in range(D // LANES):
                zbuf[r, pl.ds(j * LANES, LANES)] = zeros16

        @pl.loop(0, RPT // 16)
        def _(k):
            pltpu.sync_copy(zbuf, acc.at[pl.ds(sid * RPT + k * 16, 16)])

        plsc.subcore_barrier()

        @pl.loop(0, CPT, step=2)
        def _(c):
            @pl.when(c + 1 < CPT)
            def _():
                pltpu.sync_copy(src_hbm.at[base + c + 1], srcv1)
                pltpu.sync_copy(dst_hbm.at[base + c + 1], dstv1)
                pltpu.async_copy(y_hbm.at[srcv1], rows1, sem1)

            pltpu.make_async_copy(y_hbm.at[srcv0], rows0, sem0).wait()
            pltpu.sync_copy(rows0, acc.at[dstv0], add=True)

            @pl.when(c + 2 < CPT)
            def _():
                pltpu.sync_copy(src_hbm.at[base + c + 2], srcv0)
                pltpu.sync_copy(dst_hbm.at[base + c + 2], dstv0)
                pltpu.async_copy(y_hbm.at[srcv0], rows0, sem0)

            @pl.when(c + 1 < CPT)
            def _():
                pltpu.make_async_copy(y_hbm.at[srcv1], rows1, sem1).wait()
                pltpu.sync_copy(rows1, acc.at[dstv1], add=True)

        plsc.subcore_barrier()
        pltpu.sync_copy(acc.at[pl.ds(sid * RPT, RPT)],
                        out_hbm.at[cid, pl.ds(sid * RPT, RPT)])

    return scat_kernel(y, src2d, dst2d)


def _dinv_of(hist_blk):
    deg = jnp.sum(hist_blk, axis=1) + 1.0
    return lax.rsqrt(deg).reshape(-1, 1)


def _prep_body(x_ref, hist_ref, w1_ref, b1_ref, w2_ref, b2_ref, gw_ref, y_ref):
    dinv = _dinv_of(hist_ref[...])
    h = jnp.maximum(
        jnp.dot(x_ref[...], w1_ref[...], preferred_element_type=F32,
                precision=HIGH) + b1_ref[...], 0.0)
    h = jnp.dot(h, w2_ref[...], preferred_element_type=F32,
                precision=HIGH) + b2_ref[...]
    xw = jnp.dot(h, gw_ref[...], preferred_element_type=F32, precision=HIGH)
    y_ref[...] = xw * dinv


@jax.jit
def _prep(x, hist, w1, b1, w2, b2, gw):
    return pl.pallas_call(
        _prep_body,
        grid=(NBLK,),
        in_specs=[
            pl.BlockSpec((BR, D), lambda i: (i, 0)),
            pl.BlockSpec((BR, NW), lambda i: (i, 0)),
            pl.BlockSpec((D, H), lambda i: (0, 0)),
            pl.BlockSpec((1, H), lambda i: (0, 0)),
            pl.BlockSpec((H, H), lambda i: (0, 0)),
            pl.BlockSpec((1, H), lambda i: (0, 0)),
            pl.BlockSpec((H, H), lambda i: (0, 0)),
        ],
        out_specs=pl.BlockSpec((BR, H), lambda i: (i, 0)),
        out_shape=jax.ShapeDtypeStruct((N, H), F32),
    )(x, hist, w1, b1, w2, b2, gw)


def _mid_body(acc_ref, y_ref, hist_ref, b_ref, gw_ref, y2_ref):
    dinv = _dinv_of(hist_ref[...])
    s = acc_ref[0] + acc_ref[1] + y_ref[...]
    h = jnp.maximum(s * dinv + b_ref[...], 0.0)
    y2_ref[...] = jnp.dot(h, gw_ref[...], preferred_element_type=F32,
                          precision=HIGH) * dinv


@jax.jit
def _mid(acc, y, hist, b, gw):
    return pl.pallas_call(
        _mid_body,
        grid=(NBLK,),
        in_specs=[
            pl.BlockSpec((NC, BR, H), lambda i: (0, i, 0)),
            pl.BlockSpec((BR, H), lambda i: (i, 0)),
            pl.BlockSpec((BR, NW), lambda i: (i, 0)),
            pl.BlockSpec((1, H), lambda i: (0, 0)),
            pl.BlockSpec((H, H), lambda i: (0, 0)),
        ],
        out_specs=pl.BlockSpec((BR, H), lambda i: (i, 0)),
        out_shape=jax.ShapeDtypeStruct((N, H), F32),
    )(acc, y, hist, b, gw)


def _final_body(acc_ref, y_ref, hist_ref, batch_ref, b_ref,
                dw1_ref, db1_ref, dw2_ref, db2_ref, out_ref, pool_scr):
    i = pl.program_id(0)
    dinv = _dinv_of(hist_ref[...])
    s = acc_ref[0] + acc_ref[1] + y_ref[...]
    h = jnp.maximum(s * dinv + b_ref[...], 0.0)
    b = batch_ref[0, 0]
    oh = (b[:, None] == lax.broadcasted_iota(jnp.int32, (BR, G), 1)).astype(F32)
    part = lax.dot_general(oh, h, (((0,), (0,)), ((), ())),
                           preferred_element_type=F32, precision=HIGH)

    @pl.when(i == 0)
    def _():
        pool_scr[...] = part

    @pl.when(i > 0)
    def _():
        pool_scr[...] += part

    @pl.when(i == NBLK - 1)
    def _():
        pooled = pool_scr[...]
        d = jnp.maximum(
            jnp.dot(pooled, dw1_ref[...], preferred_element_type=F32,
                    precision=HIGH) + db1_ref[...], 0.0)
        out_ref[...] = jnp.dot(d, dw2_ref[...], preferred_element_type=F32,
                               precision=HIGH) + db2_ref[...]


@jax.jit
def _final(acc, y, hist, batch3, b, dw1, db1, dw2, db2):
    return pl.pallas_call(
        _final_body,
        grid=(NBLK,),
        in_specs=[
            pl.BlockSpec((NC, BR, H), lambda i: (0, i, 0)),
            pl.BlockSpec((BR, H), lambda i: (i, 0)),
            pl.BlockSpec((BR, NW), lambda i: (i, 0)),
            pl.BlockSpec((1, 1, BR), lambda i: (i, 0, 0)),
            pl.BlockSpec((1, H), lambda i: (0, 0)),
            pl.BlockSpec((H, H), lambda i: (0, 0)),
            pl.BlockSpec((1, H), lambda i: (0, 0)),
            pl.BlockSpec((H, OUT), lambda i: (0, 0)),
            pl.BlockSpec((1, OUT), lambda i: (0, 0)),
        ],
        out_specs=pl.BlockSpec((G, OUT), lambda i: (0, 0)),
        out_shape=jax.ShapeDtypeStruct((G, OUT), F32),
        scratch_shapes=[pltpu.VMEM((G, H), F32)],
    )(acc, y, hist, batch3, b, dw1, db1, dw2, db2)


@jax.jit
def kernel(x, edge_index, batch, enc_W1, enc_b1, enc_W2, enc_b2,
           gW0, gb0, gW1, gb1, dec_W1, dec_b1, dec_W2, dec_b2):
    src = edge_index[0]
    dst = edge_index[1]
    pad = EP - E
    src2d = jnp.concatenate(
        [src, jnp.arange(pad, dtype=jnp.int32) % N]).reshape(NW * CPT, C)
    dst2d = jnp.concatenate(
        [dst, N + (jnp.arange(pad, dtype=jnp.int32) % C)]).reshape(NW * CPT, C)
    batch3 = batch.reshape(NBLK, 1, BR)

    hist = _degree_partials(dst2d).T

    y1 = _prep(x, hist, enc_W1, enc_b1.reshape(1, H), enc_W2,
               enc_b2.reshape(1, H), gW0)
    acc1 = _edge_scatter(y1, src2d, dst2d)
    y2 = _mid(acc1, y1, hist, gb0.reshape(1, H), gW1)
    acc2 = _edge_scatter(y2, src2d, dst2d)
    return _final(acc2, y2, hist, batch3, gb1.reshape(1, H),
                  dec_W1, dec_b1.reshape(1, H), dec_W2, dec_b2.reshape(1, OUT))


# speedup vs baseline: 2.6213x; 1.0833x over previous
import dataclasses
import functools

import jax
import jax.numpy as jnp
from jax import lax
from jax.experimental import pallas as pl
from jax.experimental.pallas import tpu as pltpu
from jax.experimental.pallas import tpu_sc as plsc

N = 10000
E = 320000
D = 128
H = 128
OUT = 128
G = 64

NC = 2
NS = 16
NW = NC * NS
LANES = 16

C = 125
CPT = 80
EPT = CPT * C
NP = 10240
RPT = NP // NS
NL = 10016

BR = 1000
NBLK = N // BR

_mesh = plsc.VectorSubcoreMesh(core_axis_name="c", subcore_axis_name="s")

_sc_params = pltpu.CompilerParams()
if "needs_layout_passes" in pltpu.CompilerParams.__dataclass_fields__:
    _sc_params = dataclasses.replace(_sc_params, needs_layout_passes=False)

F32 = jnp.float32


@jax.jit
def _degree_partials(dst_flat):
    @functools.partial(
        pl.kernel,
        out_type=jax.ShapeDtypeStruct((NW, NL), F32),
        mesh=_mesh,
        compiler_params=_sc_params,
        scratch_types=[
            pltpu.VMEM((EPT,), jnp.int32),
            pltpu.VMEM((NL,), F32),
        ],
    )
    def deg_kernel(dst_hbm, out_hbm, idx_v, hist_v):
        cid = lax.axis_index("c")
        sid = lax.axis_index("s")
        wid = cid * NS + sid
        zeros16 = jnp.zeros((LANES,), F32)
        ones16 = jnp.ones((LANES,), F32)

        pltpu.sync_copy(dst_hbm.at[pl.ds(wid * EPT, EPT)], idx_v)

        @pl.loop(0, NL, step=LANES)
        def _(i):
            hist_v[pl.ds(i, LANES)] = zeros16

        @pl.loop(0, EPT, step=LANES)
        def _(c):
            iv = idx_v[pl.ds(c, LANES)]
            plsc.addupdate_scatter(hist_v, [iv], ones16)

        pltpu.sync_copy(hist_v, out_hbm.at[wid])

    return deg_kernel(dst_flat)


@jax.jit
def _edge_scatter(y, src2d, dst2d):
    @functools.partial(
        pl.kernel,
        out_type=jax.ShapeDtypeStruct((NC, NP, D), F32),
        mesh=_mesh,
        compiler_params=_sc_params,
        scratch_types=[
            pltpu.VMEM((C,), jnp.int32),
            pltpu.VMEM((C,), jnp.int32),
            pltpu.VMEM((C,), jnp.int32),
            pltpu.VMEM((C,), jnp.int32),
            pltpu.VMEM((C, D), F32),
            pltpu.VMEM((C, D), F32),
            pltpu.VMEM((16, D), F32),
            pltpu.VMEM_SHARED((NP, D), F32),
            pltpu.SemaphoreType.DMA,
            pltpu.SemaphoreType.DMA,
        ],
    )
    def scat_kernel(y_hbm, src_hbm, dst_hbm, out_hbm,
                    srcv0, dstv0, srcv1, dstv1, rows0, rows1, zbuf, acc,
                    sem0, sem1):
        cid = lax.axis_index("c")
        sid = lax.axis_index("s")
        wid = cid * NS + sid
        base = wid * CPT
        zeros16 = jnp.zeros((LANES,), F32)

        pltpu.sync_copy(src_hbm.at[base], srcv0)
        pltpu.sync_copy(dst_hbm.at[base], dstv0)
        pltpu.async_copy(y_hbm.at[srcv0], rows0, sem0)

        @pl.loop(0, 16)
        def _(r):
            for j in range(D // LANES):
                zbuf[r, pl.ds(j * LANES, LANES)] = zeros16

        @pl.loop(0, RPT // 16)
        def _(k):
            pltpu.sync_copy(zbuf, acc.at[pl.ds(sid * RPT + k * 16, 16)])

        plsc.subcore_barrier()

        @pl.loop(0, CPT, step=2)
        def _(c):
            @pl.when(c + 1 < CPT)
            def _():
                pltpu.sync_copy(src_hbm.at[base + c + 1], srcv1)
                pltpu.sync_copy(dst_hbm.at[base + c + 1], dstv1)
                pltpu.async_copy(y_hbm.at[srcv1], rows1, sem1)

            pltpu.make_async_copy(y_hbm.at[srcv0], rows0, sem0).wait()
            pltpu.sync_copy(rows0, acc.at[dstv0], add=True)

            @pl.when(c + 2 < CPT)
            def _():
                pltpu.sync_copy(src_hbm.at[base + c + 2], srcv0)
                pltpu.sync_copy(dst_hbm.at[base + c + 2], dstv0)
                pltpu.async_copy(y_hbm.at[srcv0], rows0, sem0)

            @pl.when(c + 1 < CPT)
            def _():
                pltpu.make_async_copy(y_hbm.at[srcv1], rows1, sem1).wait()
                pltpu.sync_copy(rows1, acc.at[dstv1], add=True)

        plsc.subcore_barrier()
        pltpu.sync_copy(acc.at[pl.ds(sid * RPT, RPT)],
                        out_hbm.at[cid, pl.ds(sid * RPT, RPT)])

    return scat_kernel(y, src2d, dst2d)


def _dinv_of(hist_blk):
    deg = jnp.sum(hist_blk, axis=1) + 1.0
    return lax.rsqrt(deg).reshape(-1, 1)


def _enc_body(x_ref, w1_ref, b1_ref, w2_ref, b2_ref, gw_ref, xw_ref):
    h = jnp.maximum(
        jnp.dot(x_ref[...], w1_ref[...], preferred_element_type=F32)
        + b1_ref[...], 0.0)
    h = jnp.dot(h, w2_ref[...], preferred_element_type=F32) + b2_ref[...]
    xw_ref[...] = jnp.dot(h, gw_ref[...], preferred_element_type=F32)


@jax.jit
def _enc(x, w1, b1, w2, b2, gw):
    return pl.pallas_call(
        _enc_body,
        grid=(NBLK,),
        in_specs=[
            pl.BlockSpec((BR, D), lambda i: (i, 0)),
            pl.BlockSpec((D, H), lambda i: (0, 0)),
            pl.BlockSpec((1, H), lambda i: (0, 0)),
            pl.BlockSpec((H, H), lambda i: (0, 0)),
            pl.BlockSpec((1, H), lambda i: (0, 0)),
            pl.BlockSpec((H, H), lambda i: (0, 0)),
        ],
        out_specs=pl.BlockSpec((BR, H), lambda i: (i, 0)),
        out_shape=jax.ShapeDtypeStruct((N, H), F32),
    )(x, w1, b1, w2, b2, gw)


def _scale_body(xw_ref, hist_ref, y_ref):
    y_ref[...] = xw_ref[...] * _dinv_of(hist_ref[...])


@jax.jit
def _scale(xw, hist):
    return pl.pallas_call(
        _scale_body,
        grid=(NBLK,),
        in_specs=[
            pl.BlockSpec((BR, H), lambda i: (i, 0)),
            pl.BlockSpec((BR, NW), lambda i: (i, 0)),
        ],
        out_specs=pl.BlockSpec((BR, H), lambda i: (i, 0)),
        out_shape=jax.ShapeDtypeStruct((N, H), F32),
    )(xw, hist)


def _mid_body(acc_ref, y_ref, hist_ref, b_ref, gw_ref, y2_ref):
    dinv = _dinv_of(hist_ref[...])
    s = acc_ref[0] + acc_ref[1] + y_ref[...]
    h = jnp.maximum(s * dinv + b_ref[...], 0.0)
    y2_ref[...] = jnp.dot(h, gw_ref[...], preferred_element_type=F32) * dinv


@jax.jit
def _mid(acc, y, hist, b, gw):
    return pl.pallas_call(
        _mid_body,
        grid=(NBLK,),
        in_specs=[
            pl.BlockSpec((NC, BR, H), lambda i: (0, i, 0)),
            pl.BlockSpec((BR, H), lambda i: (i, 0)),
            pl.BlockSpec((BR, NW), lambda i: (i, 0)),
            pl.BlockSpec((1, H), lambda i: (0, 0)),
            pl.BlockSpec((H, H), lambda i: (0, 0)),
        ],
        out_specs=pl.BlockSpec((BR, H), lambda i: (i, 0)),
        out_shape=jax.ShapeDtypeStruct((N, H), F32),
    )(acc, y, hist, b, gw)


def _final_body(acc_ref, y_ref, hist_ref, batch_ref, b_ref,
                dw1_ref, db1_ref, dw2_ref, db2_ref, out_ref, pool_scr):
    i = pl.program_id(0)
    dinv = _dinv_of(hist_ref[...])
    s = acc_ref[0] + acc_ref[1] + y_ref[...]
    h = jnp.maximum(s * dinv + b_ref[...], 0.0)
    b = batch_ref[0, 0]
    oh = (b[:, None] == lax.broadcasted_iota(jnp.int32, (BR, G), 1)).astype(F32)
    part = lax.dot_general(oh, h, (((0,), (0,)), ((), ())),
                           preferred_element_type=F32)

    @pl.when(i == 0)
    def _():
        pool_scr[...] = part

    @pl.when(i > 0)
    def _():
        pool_scr[...] += part

    @pl.when(i == NBLK - 1)
    def _():
        pooled = pool_scr[...]
        d = jnp.maximum(
            jnp.dot(pooled, dw1_ref[...], preferred_element_type=F32)
            + db1_ref[...], 0.0)
        out_ref[...] = (jnp.dot(d, dw2_ref[...], preferred_element_type=F32)
                        + db2_ref[...])


@jax.jit
def _final(acc, y, hist, batch3, b, dw1, db1, dw2, db2):
    return pl.pallas_call(
        _final_body,
        grid=(NBLK,),
        in_specs=[
            pl.BlockSpec((NC, BR, H), lambda i: (0, i, 0)),
            pl.BlockSpec((BR, H), lambda i: (i, 0)),
            pl.BlockSpec((BR, NW), lambda i: (i, 0)),
            pl.BlockSpec((1, 1, BR), lambda i: (i, 0, 0)),
            pl.BlockSpec((1, H), lambda i: (0, 0)),
            pl.BlockSpec((H, H), lambda i: (0, 0)),
            pl.BlockSpec((1, H), lambda i: (0, 0)),
            pl.BlockSpec((H, OUT), lambda i: (0, 0)),
            pl.BlockSpec((1, OUT), lambda i: (0, 0)),
        ],
        out_specs=pl.BlockSpec((G, OUT), lambda i: (0, 0)),
        out_shape=jax.ShapeDtypeStruct((G, OUT), F32),
        scratch_shapes=[pltpu.VMEM((G, H), F32)],
    )(acc, y, hist, batch3, b, dw1, db1, dw2, db2)


@jax.jit
def kernel(x, edge_index, batch, enc_W1, enc_b1, enc_W2, enc_b2,
           gW0, gb0, gW1, gb1, dec_W1, dec_b1, dec_W2, dec_b2):
    src = edge_index[0]
    dst = edge_index[1]
    src2d = src.reshape(NW * CPT, C)
    dst2d = dst.reshape(NW * CPT, C)
    batch3 = batch.reshape(NBLK, 1, BR)

    hist = _degree_partials(dst).T

    xw = _enc(x, enc_W1, enc_b1.reshape(1, H), enc_W2,
              enc_b2.reshape(1, H), gW0)
    y1 = _scale(xw, hist)
    acc1 = _edge_scatter(y1, src2d, dst2d)
    y2 = _mid(acc1, y1, hist, gb0.reshape(1, H), gW1)
    acc2 = _edge_scatter(y2, src2d, dst2d)
    return _final(acc2, y2, hist, batch3, gb1.reshape(1, H),
                  dec_W1, dec_b1.reshape(1, H), dec_W2, dec_b2.reshape(1, OUT))


# speedup vs baseline: 2.6930x; 1.0274x over previous
import dataclasses
import functools

import jax
import jax.numpy as jnp
from jax import lax
from jax.experimental import pallas as pl
from jax.experimental.pallas import tpu as pltpu
from jax.experimental.pallas import tpu_sc as plsc

N = 10000
E = 320000
D = 128
H = 128
OUT = 128
G = 64

NC = 2
NS = 16
NW = NC * NS
LANES = 16

C = 125
CPT = 80
EPT = CPT * C
NP = 10240
RPT = NP // NS
NL = 10016

BR = 1000
NBLK = N // BR

_mesh = plsc.VectorSubcoreMesh(core_axis_name="c", subcore_axis_name="s")

_sc_params = pltpu.CompilerParams()
if "needs_layout_passes" in pltpu.CompilerParams.__dataclass_fields__:
    _sc_params = dataclasses.replace(_sc_params, needs_layout_passes=False)

F32 = jnp.float32


@jax.jit
def _degree_partials(edge_flat):
    @functools.partial(
        pl.kernel,
        out_type=jax.ShapeDtypeStruct((NW, NL), F32),
        mesh=_mesh,
        compiler_params=_sc_params,
        scratch_types=[
            pltpu.VMEM((EPT,), jnp.int32),
            pltpu.VMEM((NL,), F32),
        ],
    )
    def deg_kernel(dst_hbm, out_hbm, idx_v, hist_v):
        cid = lax.axis_index("c")
        sid = lax.axis_index("s")
        wid = cid * NS + sid
        zeros16 = jnp.zeros((LANES,), F32)
        ones16 = jnp.ones((LANES,), F32)

        pltpu.sync_copy(dst_hbm.at[pl.ds(E + wid * EPT, EPT)], idx_v)

        @pl.loop(0, NL, step=LANES)
        def _(i):
            hist_v[pl.ds(i, LANES)] = zeros16

        @pl.loop(0, EPT, step=LANES)
        def _(c):
            iv = idx_v[pl.ds(c, LANES)]
            plsc.addupdate_scatter(hist_v, [iv], ones16)

        pltpu.sync_copy(hist_v, out_hbm.at[wid])

    return deg_kernel(edge_flat)


@jax.jit
def _edge_scatter(y, ei3):
    @functools.partial(
        pl.kernel,
        out_type=jax.ShapeDtypeStruct((NC, NP, D), F32),
        mesh=_mesh,
        compiler_params=_sc_params,
        scratch_types=[
            pltpu.VMEM((C,), jnp.int32),
            pltpu.VMEM((C,), jnp.int32),
            pltpu.VMEM((C,), jnp.int32),
            pltpu.VMEM((C,), jnp.int32),
            pltpu.VMEM((C, D), F32),
            pltpu.VMEM((C, D), F32),
            pltpu.VMEM((16, D), F32),
            pltpu.VMEM_SHARED((NP, D), F32),
            pltpu.SemaphoreType.DMA,
            pltpu.SemaphoreType.DMA,
        ],
    )
    def scat_kernel(y_hbm, ei_hbm, out_hbm,
                    srcv0, dstv0, srcv1, dstv1, rows0, rows1, zbuf, acc,
                    sem0, sem1):
        cid = lax.axis_index("c")
        sid = lax.axis_index("s")
        wid = cid * NS + sid
        base = wid * CPT
        src_hbm = ei_hbm.at[0]
        dst_hbm = ei_hbm.at[1]
        zeros16 = jnp.zeros((LANES,), F32)

        pltpu.sync_copy(src_hbm.at[base], srcv0)
        pltpu.sync_copy(dst_hbm.at[base], dstv0)
        pltpu.async_copy(y_hbm.at[srcv0], rows0, sem0)

        @pl.loop(0, 16)
        def _(r):
            for j in range(D // LANES):
                zbuf[r, pl.ds(j * LANES, LANES)] = zeros16

        @pl.loop(0, RPT // 16)
        def _(k):
            pltpu.sync_copy(zbuf, acc.at[pl.ds(sid * RPT + k * 16, 16)])

        plsc.subcore_barrier()

        @pl.loop(0, CPT, step=2)
        def _(c):
            @pl.when(c + 1 < CPT)
            def _():
                pltpu.sync_copy(src_hbm.at[base + c + 1], srcv1)
                pltpu.sync_copy(dst_hbm.at[base + c + 1], dstv1)
                pltpu.async_copy(y_hbm.at[srcv1], rows1, sem1)

            pltpu.make_async_copy(y_hbm.at[srcv0], rows0, sem0).wait()
            pltpu.sync_copy(rows0, acc.at[dstv0], add=True)

            @pl.when(c + 2 < CPT)
            def _():
                pltpu.sync_copy(src_hbm.at[base + c + 2], srcv0)
                pltpu.sync_copy(dst_hbm.at[base + c + 2], dstv0)
                pltpu.async_copy(y_hbm.at[srcv0], rows0, sem0)

            @pl.when(c + 1 < CPT)
            def _():
                pltpu.make_async_copy(y_hbm.at[srcv1], rows1, sem1).wait()
                pltpu.sync_copy(rows1, acc.at[dstv1], add=True)

        plsc.subcore_barrier()
        pltpu.sync_copy(acc.at[pl.ds(sid * RPT, RPT)],
                        out_hbm.at[cid, pl.ds(sid * RPT, RPT)])

    return scat_kernel(y, ei3)


def _dinv_of(hist_blk):
    deg = jnp.sum(hist_blk, axis=1) + 1.0
    return lax.rsqrt(deg).reshape(-1, 1)


def _enc_body(x_ref, w1_ref, b1_ref, w2_ref, b2_ref, gw_ref, xw_ref):
    h = jnp.maximum(
        jnp.dot(x_ref[...], w1_ref[...], preferred_element_type=F32)
        + b1_ref[...], 0.0)
    h = jnp.dot(h, w2_ref[...], preferred_element_type=F32) + b2_ref[...]
    xw_ref[...] = jnp.dot(h, gw_ref[...], preferred_element_type=F32)


@jax.jit
def _enc(x, w1, b1, w2, b2, gw):
    return pl.pallas_call(
        _enc_body,
        grid=(NBLK,),
        in_specs=[
            pl.BlockSpec((BR, D), lambda i: (i, 0)),
            pl.BlockSpec((D, H), lambda i: (0, 0)),
            pl.BlockSpec((1, H), lambda i: (0, 0)),
            pl.BlockSpec((H, H), lambda i: (0, 0)),
            pl.BlockSpec((1, H), lambda i: (0, 0)),
            pl.BlockSpec((H, H), lambda i: (0, 0)),
        ],
        out_specs=pl.BlockSpec((BR, H), lambda i: (i, 0)),
        out_shape=jax.ShapeDtypeStruct((N, H), F32),
    )(x, w1, b1, w2, b2, gw)


def _scale_body(xw_ref, hist_ref, y_ref):
    y_ref[...] = xw_ref[...] * _dinv_of(hist_ref[...])


@jax.jit
def _scale(xw, hist):
    return pl.pallas_call(
        _scale_body,
        grid=(NBLK,),
        in_specs=[
            pl.BlockSpec((BR, H), lambda i: (i, 0)),
            pl.BlockSpec((BR, NW), lambda i: (i, 0)),
        ],
        out_specs=pl.BlockSpec((BR, H), lambda i: (i, 0)),
        out_shape=jax.ShapeDtypeStruct((N, H), F32),
    )(xw, hist)


def _mid_body(acc_ref, y_ref, hist_ref, b_ref, gw_ref, y2_ref):
    dinv = _dinv_of(hist_ref[...])
    s = acc_ref[0] + acc_ref[1] + y_ref[...]
    h = jnp.maximum(s * dinv + b_ref[...], 0.0)
    y2_ref[...] = jnp.dot(h, gw_ref[...], preferred_element_type=F32) * dinv


@jax.jit
def _mid(acc, y, hist, b, gw):
    return pl.pallas_call(
        _mid_body,
        grid=(NBLK,),
        in_specs=[
            pl.BlockSpec((NC, BR, H), lambda i: (0, i, 0)),
            pl.BlockSpec((BR, H), lambda i: (i, 0)),
            pl.BlockSpec((BR, NW), lambda i: (i, 0)),
            pl.BlockSpec((1, H), lambda i: (0, 0)),
            pl.BlockSpec((H, H), lambda i: (0, 0)),
        ],
        out_specs=pl.BlockSpec((BR, H), lambda i: (i, 0)),
        out_shape=jax.ShapeDtypeStruct((N, H), F32),
    )(acc, y, hist, b, gw)


def _final_body(acc_ref, y_ref, hist_ref, batch_ref, b_ref,
                dw1_ref, db1_ref, dw2_ref, db2_ref, out_ref, pool_scr):
    i = pl.program_id(0)
    dinv = _dinv_of(hist_ref[...])
    s = acc_ref[0] + acc_ref[1] + y_ref[...]
    h = jnp.maximum(s * dinv + b_ref[...], 0.0)
    b = batch_ref[0, 0]
    oh = (b[:, None] == lax.broadcasted_iota(jnp.int32, (BR, G), 1)).astype(F32)
    part = lax.dot_general(oh, h, (((0,), (0,)), ((), ())),
                           preferred_element_type=F32)

    @pl.when(i == 0)
    def _():
        pool_scr[...] = part

    @pl.when(i > 0)
    def _():
        pool_scr[...] += part

    @pl.when(i == NBLK - 1)
    def _():
        pooled = pool_scr[...]
        d = jnp.maximum(
            jnp.dot(pooled, dw1_ref[...], preferred_element_type=F32)
            + db1_ref[...], 0.0)
        out_ref[...] = (jnp.dot(d, dw2_ref[...], preferred_element_type=F32)
                        + db2_ref[...])


@jax.jit
def _final(acc, y, hist, batch3, b, dw1, db1, dw2, db2):
    return pl.pallas_call(
        _final_body,
        grid=(NBLK,),
        in_specs=[
            pl.BlockSpec((NC, BR, H), lambda i: (0, i, 0)),
            pl.BlockSpec((BR, H), lambda i: (i, 0)),
            pl.BlockSpec((BR, NW), lambda i: (i, 0)),
            pl.BlockSpec((1, 1, BR), lambda i: (i, 0, 0)),
            pl.BlockSpec((1, H), lambda i: (0, 0)),
            pl.BlockSpec((H, H), lambda i: (0, 0)),
            pl.BlockSpec((1, H), lambda i: (0, 0)),
            pl.BlockSpec((H, OUT), lambda i: (0, 0)),
            pl.BlockSpec((1, OUT), lambda i: (0, 0)),
        ],
        out_specs=pl.BlockSpec((G, OUT), lambda i: (0, 0)),
        out_shape=jax.ShapeDtypeStruct((G, OUT), F32),
        scratch_shapes=[pltpu.VMEM((G, H), F32)],
    )(acc, y, hist, batch3, b, dw1, db1, dw2, db2)


@jax.jit
def kernel(x, edge_index, batch, enc_W1, enc_b1, enc_W2, enc_b2,
           gW0, gb0, gW1, gb1, dec_W1, dec_b1, dec_W2, dec_b2):
    ei3 = edge_index.reshape(2, NW * CPT, C)
    batch3 = batch.reshape(NBLK, 1, BR)

    hist = _degree_partials(edge_index.reshape(2 * E)).T

    xw = _enc(x, enc_W1, enc_b1.reshape(1, H), enc_W2,
              enc_b2.reshape(1, H), gW0)
    y1 = _scale(xw, hist)
    acc1 = _edge_scatter(y1, ei3)
    y2 = _mid(acc1, y1, hist, gb0.reshape(1, H), gW1)
    acc2 = _edge_scatter(y2, ei3)
    return _final(acc2, y2, hist, batch3, gb1.reshape(1, H),
                  dec_W1, dec_b1.reshape(1, H), dec_W2, dec_b2.reshape(1, OUT))


# speedup vs baseline: 2.7173x; 1.0090x over previous
import dataclasses
import functools

import jax
import jax.numpy as jnp
from jax import lax
from jax.experimental import pallas as pl
from jax.experimental.pallas import tpu as pltpu
from jax.experimental.pallas import tpu_sc as plsc

N = 10000
E = 320000
D = 128
H = 128
OUT = 128
G = 64

NC = 2
NS = 16
NW = NC * NS
LANES = 16

C = 125
CPT = 80
EPT = CPT * C
NP = 10112
RPT = NP // NS
NL = 10016

BR = 1000
NBLK = N // BR

_mesh = plsc.VectorSubcoreMesh(core_axis_name="c", subcore_axis_name="s")

_sc_params = pltpu.CompilerParams()
if "needs_layout_passes" in pltpu.CompilerParams.__dataclass_fields__:
    _sc_params = dataclasses.replace(_sc_params, needs_layout_passes=False)

F32 = jnp.float32


@jax.jit
def _degree_partials(edge_flat):
    @functools.partial(
        pl.kernel,
        out_type=jax.ShapeDtypeStruct((NW, NL), F32),
        mesh=_mesh,
        compiler_params=_sc_params,
        scratch_types=[
            pltpu.VMEM((EPT,), jnp.int32),
            pltpu.VMEM((NL,), F32),
        ],
    )
    def deg_kernel(dst_hbm, out_hbm, idx_v, hist_v):
        cid = lax.axis_index("c")
        sid = lax.axis_index("s")
        wid = cid * NS + sid
        zeros16 = jnp.zeros((LANES,), F32)
        ones16 = jnp.ones((LANES,), F32)

        pltpu.sync_copy(dst_hbm.at[pl.ds(E + wid * EPT, EPT)], idx_v)

        @pl.loop(0, NL, step=LANES)
        def _(i):
            hist_v[pl.ds(i, LANES)] = zeros16

        @pl.loop(0, EPT, step=LANES)
        def _(c):
            iv = idx_v[pl.ds(c, LANES)]
            plsc.addupdate_scatter(hist_v, [iv], ones16)

        pltpu.sync_copy(hist_v, out_hbm.at[wid])

    return deg_kernel(edge_flat)


@jax.jit
def _edge_scatter(y, ei3):
    @functools.partial(
        pl.kernel,
        out_type=jax.ShapeDtypeStruct((NC, NP, D), F32),
        mesh=_mesh,
        compiler_params=_sc_params,
        scratch_types=[
            pltpu.VMEM((C,), jnp.int32),
            pltpu.VMEM((C,), jnp.int32),
            pltpu.VMEM((C,), jnp.int32),
            pltpu.VMEM((C,), jnp.int32),
            pltpu.VMEM((C,), jnp.int32),
            pltpu.VMEM((C,), jnp.int32),
            pltpu.VMEM((C, D), F32),
            pltpu.VMEM((C, D), F32),
            pltpu.VMEM((C, D), F32),
            pltpu.VMEM_SHARED((NP, D), F32),
            pltpu.SemaphoreType.DMA,
            pltpu.SemaphoreType.DMA,
            pltpu.SemaphoreType.DMA,
        ],
    )
    def scat_kernel(y_hbm, ei_hbm, out_hbm,
                    srcv0, dstv0, srcv1, dstv1, srcv2, dstv2,
                    rows0, rows1, rows2, acc, sem0, sem1, sem2):
        cid = lax.axis_index("c")
        sid = lax.axis_index("s")
        wid = cid * NS + sid
        base = wid * CPT
        src_hbm = ei_hbm.at[0]
        dst_hbm = ei_hbm.at[1]
        srcv = (srcv0, srcv1, srcv2)
        dstv = (dstv0, dstv1, dstv2)
        rows = (rows0, rows1, rows2)
        sem = (sem0, sem1, sem2)
        zeros16 = jnp.zeros((LANES,), F32)

        for k in (0, 1):
            pltpu.sync_copy(src_hbm.at[base + k], srcv[k])
            pltpu.sync_copy(dst_hbm.at[base + k], dstv[k])
            pltpu.async_copy(y_hbm.at[srcv[k]], rows[k], sem[k])

        @pl.loop(0, C)
        def _(r):
            for j in range(D // LANES):
                rows2[r, pl.ds(j * LANES, LANES)] = zeros16

        for k in range(RPT // 120):
            pltpu.sync_copy(rows2.at[pl.ds(0, 120)],
                            acc.at[pl.ds(sid * RPT + k * 120, 120)])
        pltpu.sync_copy(rows2.at[pl.ds(0, RPT - (RPT // 120) * 120)],
                        acc.at[pl.ds(sid * RPT + (RPT // 120) * 120,
                                     RPT - (RPT // 120) * 120)])

        plsc.subcore_barrier()

        pltpu.sync_copy(src_hbm.at[base + 2], srcv2)
        pltpu.sync_copy(dst_hbm.at[base + 2], dstv2)
        pltpu.async_copy(y_hbm.at[srcv2], rows2, sem2)

        @pl.loop(0, CPT, step=3)
        def _(c):
            for k in range(3):
                @pl.when(c + k < CPT)
                def _():
                    pltpu.make_async_copy(y_hbm.at[srcv[k]], rows[k],
                                          sem[k]).wait()
                    pltpu.sync_copy(rows[k], acc.at[dstv[k]], add=True)

                    @pl.when(c + k + 3 < CPT)
                    def _():
                        pltpu.sync_copy(src_hbm.at[base + c + k + 3], srcv[k])
                        pltpu.sync_copy(dst_hbm.at[base + c + k + 3], dstv[k])
                        pltpu.async_copy(y_hbm.at[srcv[k]], rows[k], sem[k])

        plsc.subcore_barrier()
        pltpu.sync_copy(acc.at[pl.ds(sid * RPT, RPT)],
                        out_hbm.at[cid, pl.ds(sid * RPT, RPT)])

    return scat_kernel(y, ei3)


def _dinv_of(hist_blk):
    deg = jnp.sum(hist_blk, axis=1) + 1.0
    return lax.rsqrt(deg).reshape(-1, 1)


def _enc_body(x_ref, w1_ref, b1_ref, w2_ref, b2_ref, gw_ref, xw_ref):
    h = jnp.maximum(
        jnp.dot(x_ref[...], w1_ref[...], preferred_element_type=F32)
        + b1_ref[...], 0.0)
    h = jnp.dot(h, w2_ref[...], preferred_element_type=F32) + b2_ref[...]
    xw_ref[...] = jnp.dot(h, gw_ref[...], preferred_element_type=F32)


@jax.jit
def _enc(x, w1, b1, w2, b2, gw):
    return pl.pallas_call(
        _enc_body,
        grid=(NBLK,),
        in_specs=[
            pl.BlockSpec((BR, D), lambda i: (i, 0)),
            pl.BlockSpec((D, H), lambda i: (0, 0)),
            pl.BlockSpec((1, H), lambda i: (0, 0)),
            pl.BlockSpec((H, H), lambda i: (0, 0)),
            pl.BlockSpec((1, H), lambda i: (0, 0)),
            pl.BlockSpec((H, H), lambda i: (0, 0)),
        ],
        out_specs=pl.BlockSpec((BR, H), lambda i: (i, 0)),
        out_shape=jax.ShapeDtypeStruct((N, H), F32),
    )(x, w1, b1, w2, b2, gw)


def _scale_body(xw_ref, hist_ref, y_ref):
    y_ref[...] = xw_ref[...] * _dinv_of(hist_ref[...])


@jax.jit
def _scale(xw, hist):
    return pl.pallas_call(
        _scale_body,
        grid=(NBLK,),
        in_specs=[
            pl.BlockSpec((BR, H), lambda i: (i, 0)),
            pl.BlockSpec((BR, NW), lambda i: (i, 0)),
        ],
        out_specs=pl.BlockSpec((BR, H), lambda i: (i, 0)),
        out_shape=jax.ShapeDtypeStruct((N, H), F32),
    )(xw, hist)


def _mid_body(acc_ref, y_ref, hist_ref, b_ref, gw_ref, y2_ref):
    dinv = _dinv_of(hist_ref[...])
    s = acc_ref[0] + acc_ref[1] + y_ref[...]
    h = jnp.maximum(s * dinv + b_ref[...], 0.0)
    y2_ref[...] = jnp.dot(h, gw_ref[...], preferred_element_type=F32) * dinv


@jax.jit
def _mid(acc, y, hist, b, gw):
    return pl.pallas_call(
        _mid_body,
        grid=(NBLK,),
        in_specs=[
            pl.BlockSpec((NC, BR, H), lambda i: (0, i, 0)),
            pl.BlockSpec((BR, H), lambda i: (i, 0)),
            pl.BlockSpec((BR, NW), lambda i: (i, 0)),
            pl.BlockSpec((1, H), lambda i: (0, 0)),
            pl.BlockSpec((H, H), lambda i: (0, 0)),
        ],
        out_specs=pl.BlockSpec((BR, H), lambda i: (i, 0)),
        out_shape=jax.ShapeDtypeStruct((N, H), F32),
    )(acc, y, hist, b, gw)


def _final_body(acc_ref, y_ref, hist_ref, batch_ref, b_ref,
                dw1_ref, db1_ref, dw2_ref, db2_ref, out_ref, pool_scr):
    i = pl.program_id(0)
    dinv = _dinv_of(hist_ref[...])
    s = acc_ref[0] + acc_ref[1] + y_ref[...]
    h = jnp.maximum(s * dinv + b_ref[...], 0.0)
    b = batch_ref[0, 0]
    oh = (b[:, None] == lax.broadcasted_iota(jnp.int32, (BR, G), 1)).astype(F32)
    part = lax.dot_general(oh, h, (((0,), (0,)), ((), ())),
                           preferred_element_type=F32)

    @pl.when(i == 0)
    def _():
        pool_scr[...] = part

    @pl.when(i > 0)
    def _():
        pool_scr[...] += part

    @pl.when(i == NBLK - 1)
    def _():
        pooled = pool_scr[...]
        d = jnp.maximum(
            jnp.dot(pooled, dw1_ref[...], preferred_element_type=F32)
            + db1_ref[...], 0.0)
        out_ref[...] = (jnp.dot(d, dw2_ref[...], preferred_element_type=F32)
                        + db2_ref[...])


@jax.jit
def _final(acc, y, hist, batch3, b, dw1, db1, dw2, db2):
    return pl.pallas_call(
        _final_body,
        grid=(NBLK,),
        in_specs=[
            pl.BlockSpec((NC, BR, H), lambda i: (0, i, 0)),
            pl.BlockSpec((BR, H), lambda i: (i, 0)),
            pl.BlockSpec((BR, NW), lambda i: (i, 0)),
            pl.BlockSpec((1, 1, BR), lambda i: (i, 0, 0)),
            pl.BlockSpec((1, H), lambda i: (0, 0)),
            pl.BlockSpec((H, H), lambda i: (0, 0)),
            pl.BlockSpec((1, H), lambda i: (0, 0)),
            pl.BlockSpec((H, OUT), lambda i: (0, 0)),
            pl.BlockSpec((1, OUT), lambda i: (0, 0)),
        ],
        out_specs=pl.BlockSpec((G, OUT), lambda i: (0, 0)),
        out_shape=jax.ShapeDtypeStruct((G, OUT), F32),
        scratch_shapes=[pltpu.VMEM((G, H), F32)],
    )(acc, y, hist, batch3, b, dw1, db1, dw2, db2)


@jax.jit
def kernel(x, edge_index, batch, enc_W1, enc_b1, enc_W2, enc_b2,
           gW0, gb0, gW1, gb1, dec_W1, dec_b1, dec_W2, dec_b2):
    ei3 = edge_index.reshape(2, NW * CPT, C)
    batch3 = batch.reshape(NBLK, 1, BR)

    hist = _degree_partials(edge_index.reshape(2 * E)).T

    xw = _enc(x, enc_W1, enc_b1.reshape(1, H), enc_W2,
              enc_b2.reshape(1, H), gW0)
    y1 = _scale(xw, hist)
    acc1 = _edge_scatter(y1, ei3)
    y2 = _mid(acc1, y1, hist, gb0.reshape(1, H), gW1)
    acc2 = _edge_scatter(y2, ei3)
    return _final(acc2, y2, hist, batch3, gb1.reshape(1, H),
                  dec_W1, dec_b1.reshape(1, H), dec_W2, dec_b2.reshape(1, OUT))
